# baseline (speedup 1.0000x reference)
import functools
import math

import jax
import jax.numpy as jnp
from jax import lax
from jax.experimental import pallas as pl
from jax.experimental.pallas import tpu as pltpu
from jax.experimental.pallas import tpu_sc as plsc

VOCAB = 1000000
D = 64
B_TOTAL = 16384 * 50
NC, NS = 2, 16
NW = NC * NS
B_PER_W = B_TOTAL // NW
CHUNK = 1024
GRP = 128
G = CHUNK // GRP
N_CHUNKS = B_PER_W // CHUNK
U = 8
SCALE = math.sqrt(D)


def _emb_kernel(w_hbm, idx_hbm, out_hbm, idx_v, rows_v, gsem):
    wid = lax.axis_index("s") * NC + lax.axis_index("c")
    base = wid * B_PER_W
    grow0 = base // GRP

    def chunk_body(ci, carry):
        goff = pl.multiple_of(grow0 + ci * G, 8)
        pltpu.sync_copy(idx_hbm.at[pl.ds(goff, G)], idx_v)
        descs = [
            pltpu.async_copy(
                w_hbm.at[idx_v.at[g]], rows_v.at[pl.ds(g * GRP, GRP)], gsem
            )
            for g in range(G)
        ]
        for d in descs:
            d.wait()

        def scale_body(ri, c):
            for u in range(U):
                r = ri * U + u
                for j in range(D // 16):
                    sl = pl.ds(j * 16, 16)
                    rows_v[r, sl] = rows_v[r, sl] * SCALE
            return c

        lax.fori_loop(0, CHUNK // U, scale_body, 0, unroll=False)

        ooff = pl.multiple_of(base + ci * CHUNK, 8)
        pltpu.sync_copy(rows_v, out_hbm.at[pl.ds(ooff, CHUNK)])
        return carry

    lax.fori_loop(0, N_CHUNKS, chunk_body, 0, unroll=False)


@jax.jit
def _emb(weight, idx2d):
    mesh = plsc.VectorSubcoreMesh(
        core_axis_name="c", subcore_axis_name="s", num_cores=NC, num_subcores=NS
    )
    run = pl.kernel(
        _emb_kernel,
        out_type=jax.ShapeDtypeStruct((B_TOTAL, D), jnp.float32),
        mesh=mesh,
        scratch_types=[
            pltpu.VMEM((G, GRP), jnp.int32),
            pltpu.VMEM((CHUNK, D), jnp.float32),
            pltpu.SemaphoreType.DMA,
        ],
        compiler_params=pltpu.CompilerParams(use_tc_tiling_on_sc=False),
    )
    return run(weight, idx2d)


def kernel(input_ids, weight):
    idx2d = input_ids.reshape(B_TOTAL // GRP, GRP).astype(jnp.int32)
    out = _emb(weight, idx2d)
    return out.reshape(input_ids.shape + (D,))


# speedup vs baseline: 1.0615x; 1.0615x over previous
import math

import jax
import jax.numpy as jnp
from jax import lax
from jax.experimental import pallas as pl
from jax.experimental.pallas import tpu as pltpu
from jax.experimental.pallas import tpu_sc as plsc

VOCAB = 1000000
D = 64
B_TOTAL = 16384 * 50
NC, NS = 2, 16
NW = NC * NS
B_PER_W = B_TOTAL // NW
CHUNK = 512
GRP = 128
G = CHUNK // GRP
N_CHUNKS = B_PER_W // CHUNK
IDX_ROWS = B_PER_W // GRP
U = 8
SCALE = math.sqrt(D)


def _emb_kernel(w_hbm, idx_hbm, out_hbm, idx_all, rows, gs0, gs1, os0, os1):
    wid = lax.axis_index("s") * NC + lax.axis_index("c")
    base = wid * B_PER_W
    grow0 = pl.multiple_of(base // GRP, 8)
    gsems = (gs0, gs1)
    osems = (os0, os1)

    pltpu.sync_copy(idx_hbm.at[pl.ds(grow0, IDX_ROWS)], idx_all)

    def fire_gather(g, b):
        for k in range(G):
            pltpu.async_copy(
                w_hbm.at[idx_all.at[g * G + k]],
                rows.at[b, pl.ds(k * GRP, GRP)],
                gsems[b],
            )

    def wait_gather(b):
        pltpu.make_async_copy(
            w_hbm.at[pl.ds(0, CHUNK)], rows.at[b], gsems[b]
        ).wait()

    def scale_chunk(b):
        def body(ri, c):
            for u in range(U):
                r = ri * U + u
                for j in range(D // 16):
                    sl = pl.ds(j * 16, 16)
                    rows[b, r, sl] = rows[b, r, sl] * SCALE
            return c

        lax.fori_loop(0, CHUNK // U, body, 0, unroll=False)

    def out_desc(g, b):
        ooff = pl.multiple_of(base + g * CHUNK, 8)
        return pltpu.make_async_copy(
            rows.at[b], out_hbm.at[pl.ds(ooff, CHUNK)], osems[b]
        )

    fire_gather(0, 0)
    fire_gather(1, 1)

    @pl.loop(0, N_CHUNKS - 2, step=2)
    def _(go):
        for b in range(2):
            g = go + b
            wait_gather(b)
            scale_chunk(b)
            od = out_desc(g, b)
            od.start()
            od.wait()
            fire_gather(g + 2, b)

    for b in range(2):
        g = N_CHUNKS - 2 + b
        wait_gather(b)
        scale_chunk(b)
        od = out_desc(g, b)
        od.start()
        od.wait()


@jax.jit
def _emb(weight, idx2d):
    mesh = plsc.VectorSubcoreMesh(
        core_axis_name="c", subcore_axis_name="s", num_cores=NC, num_subcores=NS
    )
    run = pl.kernel(
        _emb_kernel,
        out_type=jax.ShapeDtypeStruct((B_TOTAL, D), jnp.float32),
        mesh=mesh,
        scratch_types=[
            pltpu.VMEM((IDX_ROWS, GRP), jnp.int32),
            pltpu.VMEM((2, CHUNK, D), jnp.float32),
            pltpu.SemaphoreType.DMA,
            pltpu.SemaphoreType.DMA,
            pltpu.SemaphoreType.DMA,
            pltpu.SemaphoreType.DMA,
        ],
        compiler_params=pltpu.CompilerParams(use_tc_tiling_on_sc=False),
    )
    return run(weight, idx2d)


def kernel(input_ids, weight):
    idx2d = input_ids.reshape(B_TOTAL // GRP, GRP).astype(jnp.int32)
    out = _emb(weight, idx2d)
    return out.reshape(input_ids.shape + (D,))
